# baseline (speedup 1.0000x reference)
import functools

import jax
import jax.numpy as jnp
from jax import lax
from jax.experimental import pallas as pl
from jax.experimental.pallas import tpu as pltpu
from jax.experimental.pallas import tpu_sc as plsc

N = 10000
E = 320000
D = 128
B = 64

NC = 2
NS = 16
NW = NC * NS

NP = 10112
PAD_NODE = 10000
CH = 128
NCHUNK = 80
EPW = NCHUNK * CH
E_PAD = EPW * NW
RPT = NP // NS
NB = 2
NIB = 4
C0 = 152
C1 = 2 * NCHUNK - C0

_mesh = plsc.VectorSubcoreMesh(core_axis_name="c", subcore_axis_name="s")

_f32 = jnp.float32


def _deg_body(dst2_hbm, ones_hbm, zeros_hbm, deg0_hbm, deg1_hbm,
              dsts, ones_rows, sem, deg_sh):
    cid = lax.axis_index("c")
    sid = lax.axis_index("s")
    wid = cid * NS + sid
    row0 = sid * RPT

    pltpu.sync_copy(ones_hbm, ones_rows)
    pltpu.sync_copy(dst2_hbm.at[pl.ds(wid * NCHUNK, NCHUNK)], dsts)
    pltpu.sync_copy(zeros_hbm.at[pl.ds(row0, RPT)],
                    deg_sh.at[pl.ds(row0, RPT)])
    plsc.subcore_barrier()

    FD = 16

    @pl.loop(0, NCHUNK // FD)
    def _grp(g):
        @pl.loop(0, FD)
        def _fire(k):
            pltpu.async_copy(ones_rows, deg_sh.at[dsts.at[g * FD + k]],
                             sem, add=True)

        @pl.loop(0, FD)
        def _drain(k):
            pltpu.make_async_copy(ones_hbm, ones_rows, sem).wait()

    plsc.subcore_barrier()

    out = (deg0_hbm, deg1_hbm)
    for c in range(NC):
        @pl.when(cid == c)
        def _():
            pltpu.sync_copy(deg_sh.at[pl.ds(row0, RPT)],
                            out[c].at[pl.ds(row0, RPT)])


_deg_kernel = functools.partial(
    pl.kernel, _deg_body, mesh=_mesh,
    out_type=[jax.ShapeDtypeStruct((NP, 128), _f32)] * 2,
    scratch_types=[
        pltpu.VMEM((NCHUNK, CH), jnp.int32),
        pltpu.VMEM((CH, 128), _f32),
        pltpu.SemaphoreType.DMA,
        pltpu.VMEM_SHARED((NP, 128), _f32),
    ],
)()


def _agg_body(src_hbm, dst_hbm, y_hbm, zeros_hbm, t0_hbm, t1_hbm,
              srcv, dstv, bufs, isems, gsems, ssems, t_sh):
    cid = lax.axis_index("c")
    sid = lax.axis_index("s")
    wid = cid * NS + sid
    row0 = sid * RPT

    def wait_idx(slot):
        pltpu.make_async_copy(src_hbm.at[pl.ds(0, CH)],
                              srcv.at[slot], isems[slot]).wait()
        pltpu.make_async_copy(src_hbm.at[pl.ds(0, CH)],
                              dstv.at[slot], isems[slot]).wait()

    def wait_buf(slot, sem):
        pltpu.make_async_copy(zeros_hbm.at[pl.ds(0, CH)],
                              bufs[slot], sem).wait()

    SUB = 1
    SR = CH // SUB

    def gather_chunk(isl, r):
        for q in range(SUB):
            pltpu.async_copy(y_hbm.at[srcv.at[isl, pl.ds(q * SR, SR)]],
                             bufs[r].at[pl.ds(q * SR, SR)], gsems[r])

    init_src = (zeros_hbm, y_hbm)
    for c in range(NC):
        @pl.when(cid == c)
        def _():
            pltpu.sync_copy(init_src[c].at[pl.ds(row0, RPT)],
                            t_sh.at[pl.ds(row0, RPT)])

    plsc.subcore_barrier()

    def run_chunks(cbase, cn):
        def load_idx(slot, j):
            off = (cbase + j) * CH
            pltpu.async_copy(src_hbm.at[pl.ds(off, CH)],
                             srcv.at[slot], isems[slot])
            pltpu.async_copy(dst_hbm.at[pl.ds(off, CH)],
                             dstv.at[slot], isems[slot])

        for s in range(NIB):
            load_idx(s, s)
        for r in range(NB):
            wait_idx(r)
            gather_chunk(r, r)

        @pl.loop(0, cn // NIB)
        def _grp(g):
            for k in range(NIB):
                j = g * NIB + k
                r = k % NB
                wait_buf(r, gsems[r])
                pltpu.async_copy(bufs[r], t_sh.at[dstv.at[k]],
                                 ssems[r], add=True)
                nj = j + NB

                @pl.when(nj < cn)
                def _():
                    wait_buf(r, ssems[r])
                    wait_idx((k + NB) % NIB)
                    gather_chunk((k + NB) % NIB, r)
                    nl = j + NIB

                    @pl.when(nl < cn)
                    def _():
                        load_idx(k, nl)

        for r in range(NB):
            wait_buf(r, ssems[r])

    if C0 > 0:
        @pl.when(cid == 0)
        def _():
            run_chunks(sid * C0, C0)

    if C1 > 0:
        @pl.when(cid == 1)
        def _():
            run_chunks(NS * C0 + sid * C1, C1)

    plsc.subcore_barrier()

    out = (t0_hbm, t1_hbm)
    for c in range(NC):
        @pl.when(cid == c)
        def _():
            pltpu.sync_copy(t_sh.at[pl.ds(row0, RPT)],
                            out[c].at[pl.ds(row0, RPT)])


_agg_kernel = functools.partial(
    pl.kernel, _agg_body, mesh=_mesh,
    out_type=[jax.ShapeDtypeStruct((NP, 128), _f32)] * 2,
    scratch_types=[
        pltpu.VMEM((NIB, CH), jnp.int32),
        pltpu.VMEM((NIB, CH), jnp.int32),
        [pltpu.VMEM((CH, 128), _f32)] * NB,
        [pltpu.SemaphoreType.DMA] * NIB,
        [pltpu.SemaphoreType.DMA] * NB,
        [pltpu.SemaphoreType.DMA] * NB,
        pltpu.VMEM_SHARED((NP, 128), _f32),
    ],
)()


def _prescale_body(x_ref, d0_ref, d1_ref, y_ref, isr_ref):
    deg = d0_ref[...] + d1_ref[...] + 1.0
    rows = lax.broadcasted_iota(jnp.int32, (NP, 128), 0)
    isr = jnp.where(rows < N, lax.rsqrt(deg), 0.0)
    isr_ref[...] = isr
    y_ref[...] = x_ref[...] * isr


_prescale = pl.pallas_call(
    _prescale_body,
    out_shape=[jax.ShapeDtypeStruct((NP, 128), _f32),
               jax.ShapeDtypeStruct((NP, 128), _f32)],
)


def _layer_body(t0_ref, t1_ref, isr_ref, w_ref, b_ref, y_ref):
    isr = isr_ref[...]
    h = (t0_ref[...] + t1_ref[...]) * isr
    x = jax.nn.relu(jnp.dot(h, w_ref[...], preferred_element_type=_f32)
                    + b_ref[...])
    y_ref[...] = x * isr


_layer = pl.pallas_call(
    _layer_body,
    out_shape=jax.ShapeDtypeStruct((NP, 128), _f32),
)


def _final_body(t0_ref, t1_ref, isr_ref, w_ref, b_ref, gid_ref, xa_ref,
                wd1a_ref, wd1b_ref, bd1_ref, wd2_ref, bd2_ref,
                wout_ref, bout_ref, out_ref):
    h = (t0_ref[...] + t1_ref[...]) * isr_ref[...]
    x3 = jax.nn.relu(jnp.dot(h, w_ref[...], preferred_element_type=_f32)
                     + b_ref[...])
    gsel = lax.broadcasted_iota(jnp.int32, (B, NP), 0)
    mask = jnp.where(gsel == gid_ref[...], 1.0, 0.0)
    r = jnp.dot(mask, x3, preferred_element_type=_f32,
                precision=lax.Precision.HIGHEST)
    h1 = jax.nn.relu(
        jnp.dot(r, wd1a_ref[...], preferred_element_type=_f32)
        + jnp.dot(xa_ref[...], wd1b_ref[...], preferred_element_type=_f32)
        + bd1_ref[...])
    h2 = jax.nn.relu(jnp.dot(h1, wd2_ref[...], preferred_element_type=_f32)
                     + bd2_ref[...])
    out_ref[...] = (jnp.dot(h2, wout_ref[...], preferred_element_type=_f32)
                    + bout_ref[...])


_final = pl.pallas_call(
    _final_body,
    out_shape=jax.ShapeDtypeStruct((B, 1), _f32),
)


def kernel(x_mol, edge_index, graph_ids, x_adduct,
           Wg1, bg1, Wg2, bg2, Wg3, bg3,
           Wd1, bd1, Wd2, bd2, Wout, bout):
    x_pad = jnp.pad(x_mol, ((0, NP - N), (0, 0)))
    src = jnp.pad(edge_index[0], (0, E_PAD - E), constant_values=PAD_NODE)
    dst = jnp.pad(edge_index[1], (0, E_PAD - E), constant_values=PAD_NODE)
    gid = jnp.pad(graph_ids, (0, NP - N), constant_values=B)[None, :]
    zeros = jnp.zeros((NP, 128), dtype=_f32)

    dst2 = dst.reshape(E_PAD // CH, CH)
    ones2d = jnp.ones((CH, 128), dtype=_f32)
    deg0, deg1 = _deg_kernel(dst2, ones2d, zeros)
    y1, isr = _prescale(x_pad, deg0, deg1)

    t0, t1 = _agg_kernel(src, dst, y1, zeros)
    y2 = _layer(t0, t1, isr, Wg1, bg1[None, :])
    t0, t1 = _agg_kernel(src, dst, y2, zeros)
    y3 = _layer(t0, t1, isr, Wg2, bg2[None, :])
    t0, t1 = _agg_kernel(src, dst, y3, zeros)

    return _final(t0, t1, isr, Wg3, bg3[None, :], gid,
                  x_adduct.astype(_f32),
                  Wd1[:D], Wd1[D:], bd1[None, :],
                  Wd2, bd2[None, :], Wout, bout[None, :])


# speedup vs baseline: 1.4701x; 1.4701x over previous
import functools

import jax
import jax.numpy as jnp
from jax import lax
from jax.experimental import pallas as pl
from jax.experimental.pallas import tpu as pltpu
from jax.experimental.pallas import tpu_sc as plsc

N = 10000
E = 320000
D = 128
B = 64

NC = 2
NS = 16
NW = NC * NS

NP = 10112
PAD_NODE = 10000
CH = 128
NCHUNK = 80
EPW = NCHUNK * CH
E_PAD = EPW * NW
RPT = NP // NS
NB = 2
NIB = 4
C0 = 152
C1 = 2 * NCHUNK - C0
REP = 8

_mesh = plsc.VectorSubcoreMesh(core_axis_name="c", subcore_axis_name="s")

_f32 = jnp.float32


def _deg_body(dst2_hbm, ones_hbm, zeros_hbm, deg0_hbm, deg1_hbm,
              dsts, ones_rows, sem, deg_sh):
    cid = lax.axis_index("c")
    sid = lax.axis_index("s")
    wid = cid * NS + sid
    row0 = sid * RPT

    pltpu.sync_copy(ones_hbm, ones_rows)
    pltpu.sync_copy(dst2_hbm.at[pl.ds(wid * NCHUNK, NCHUNK)], dsts)
    pltpu.sync_copy(zeros_hbm.at[pl.ds(row0, RPT)],
                    deg_sh.at[pl.ds(row0, RPT)])
    plsc.subcore_barrier()

    FD = 16

    @pl.loop(0, NCHUNK // FD)
    def _grp(g):
        @pl.loop(0, FD)
        def _fire(k):
            pltpu.async_copy(ones_rows, deg_sh.at[dsts.at[g * FD + k]],
                             sem, add=True)

        @pl.loop(0, FD)
        def _drain(k):
            pltpu.make_async_copy(ones_hbm, ones_rows, sem).wait()

    plsc.subcore_barrier()

    out = (deg0_hbm, deg1_hbm)
    for c in range(NC):
        @pl.when(cid == c)
        def _():
            pltpu.sync_copy(deg_sh.at[pl.ds(row0, RPT)],
                            out[c].at[pl.ds(row0, RPT)])


_deg_kernel = functools.partial(
    pl.kernel, _deg_body, mesh=_mesh,
    out_type=[jax.ShapeDtypeStruct((NP, 128), _f32)] * 2,
    scratch_types=[
        pltpu.VMEM((NCHUNK, CH), jnp.int32),
        pltpu.VMEM((CH, 128), _f32),
        pltpu.SemaphoreType.DMA,
        pltpu.VMEM_SHARED((NP, 128), _f32),
    ],
)()


def _agg_body(src_hbm, dst_hbm, y_hbm, zeros_hbm, t0_hbm, t1_hbm,
              srcv, dstv, bufs, isems, gsems, ssems, t_sh):
    cid = lax.axis_index("c")
    sid = lax.axis_index("s")
    wid = cid * NS + sid
    row0 = sid * RPT

    def wait_idx(slot):
        pltpu.make_async_copy(src_hbm.at[pl.ds(0, CH)],
                              srcv.at[slot], isems[slot]).wait()
        pltpu.make_async_copy(src_hbm.at[pl.ds(0, CH)],
                              dstv.at[slot], isems[slot]).wait()

    def wait_buf(slot, sem):
        pltpu.make_async_copy(zeros_hbm.at[pl.ds(0, CH)],
                              bufs[slot], sem).wait()

    SUB = 4
    SR = CH // SUB

    def gather_chunk(isl, r):
        for q in range(SUB):
            pltpu.async_copy(y_hbm.at[srcv.at[isl, pl.ds(q * SR, SR)]],
                             bufs[r].at[pl.ds(q * SR, SR)], gsems[r])

    init_src = (zeros_hbm, y_hbm)
    for c in range(NC):
        @pl.when(cid == c)
        def _():
            pltpu.sync_copy(init_src[c].at[pl.ds(row0, RPT)],
                            t_sh.at[pl.ds(row0, RPT)])

    plsc.subcore_barrier()

    def run_chunks(cbase, cn):
        def load_idx(slot, j):
            off = (cbase + j) * CH
            pltpu.async_copy(src_hbm.at[pl.ds(off, CH)],
                             srcv.at[slot], isems[slot])
            pltpu.async_copy(dst_hbm.at[pl.ds(off, CH)],
                             dstv.at[slot], isems[slot])

        for s in range(NIB):
            load_idx(s, s)
        for r in range(NB):
            wait_idx(r)
            gather_chunk(r, r)

        @pl.loop(0, cn // NIB)
        def _grp(g):
            for k in range(NIB):
                j = g * NIB + k
                r = k % NB
                wait_buf(r, gsems[r])
                pltpu.async_copy(bufs[r], t_sh.at[dstv.at[k]],
                                 ssems[r], add=True)
                nj = j + NB

                @pl.when(nj < cn)
                def _():
                    wait_buf(r, ssems[r])
                    wait_idx((k + NB) % NIB)
                    gather_chunk((k + NB) % NIB, r)
                    nl = j + NIB

                    @pl.when(nl < cn)
                    def _():
                        load_idx(k, nl)

        for r in range(NB):
            wait_buf(r, ssems[r])

    if C0 > 0:
        @pl.when(cid == 0)
        def _():
            run_chunks(sid * C0, C0)

    if C1 > 0:
        @pl.when(cid == 1)
        def _():
            run_chunks(NS * C0 + sid * C1, C1)

    plsc.subcore_barrier()

    out = (t0_hbm, t1_hbm)
    for c in range(NC):
        @pl.when(cid == c)
        def _():
            pltpu.sync_copy(t_sh.at[pl.ds(row0, RPT)],
                            out[c].at[pl.ds(row0, RPT)])


_agg_kernel = functools.partial(
    pl.kernel, _agg_body, mesh=_mesh,
    out_type=[jax.ShapeDtypeStruct((NP, 128), _f32)] * 2,
    scratch_types=[
        pltpu.VMEM((NIB, CH), jnp.int32),
        pltpu.VMEM((NIB, CH), jnp.int32),
        [pltpu.VMEM((CH, 128), _f32)] * NB,
        [pltpu.SemaphoreType.DMA] * NIB,
        [pltpu.SemaphoreType.DMA] * NB,
        [pltpu.SemaphoreType.DMA] * NB,
        pltpu.VMEM_SHARED((NP, 128), _f32),
    ],
)()


def _rep_body(y_ref, out_ref):
    out_ref[...] = y_ref[...]


_replicate = pl.pallas_call(
    _rep_body,
    grid=(REP,),
    in_specs=[pl.BlockSpec((NP, 128), lambda k: (0, 0))],
    out_specs=pl.BlockSpec((NP, 128), lambda k: (k, 0)),
    out_shape=jax.ShapeDtypeStruct((REP * NP, 128), _f32),
)


def _prescale_body(x_ref, d0_ref, d1_ref, y_ref, isr_ref):
    deg = d0_ref[...] + d1_ref[...] + 1.0
    rows = lax.broadcasted_iota(jnp.int32, (NP, 128), 0)
    isr = jnp.where(rows < N, lax.rsqrt(deg), 0.0)
    isr_ref[...] = isr
    y_ref[...] = x_ref[...] * isr


_prescale = pl.pallas_call(
    _prescale_body,
    out_shape=[jax.ShapeDtypeStruct((NP, 128), _f32),
               jax.ShapeDtypeStruct((NP, 128), _f32)],
)


def _layer_body(t0_ref, t1_ref, isr_ref, w_ref, b_ref, y_ref):
    isr = isr_ref[...]
    h = (t0_ref[...] + t1_ref[...]) * isr
    x = jax.nn.relu(jnp.dot(h, w_ref[...], preferred_element_type=_f32)
                    + b_ref[...])
    y_ref[...] = x * isr


_layer = pl.pallas_call(
    _layer_body,
    out_shape=jax.ShapeDtypeStruct((NP, 128), _f32),
)


def _final_body(t0_ref, t1_ref, isr_ref, w_ref, b_ref, gid_ref, xa_ref,
                wd1a_ref, wd1b_ref, bd1_ref, wd2_ref, bd2_ref,
                wout_ref, bout_ref, out_ref):
    h = (t0_ref[...] + t1_ref[...]) * isr_ref[...]
    x3 = jax.nn.relu(jnp.dot(h, w_ref[...], preferred_element_type=_f32)
                     + b_ref[...])
    gsel = lax.broadcasted_iota(jnp.int32, (B, NP), 0)
    mask = jnp.where(gsel == gid_ref[...], 1.0, 0.0)
    r = jnp.dot(mask, x3, preferred_element_type=_f32,
                precision=lax.Precision.HIGHEST)
    h1 = jax.nn.relu(
        jnp.dot(r, wd1a_ref[...], preferred_element_type=_f32)
        + jnp.dot(xa_ref[...], wd1b_ref[...], preferred_element_type=_f32)
        + bd1_ref[...])
    h2 = jax.nn.relu(jnp.dot(h1, wd2_ref[...], preferred_element_type=_f32)
                     + bd2_ref[...])
    out_ref[...] = (jnp.dot(h2, wout_ref[...], preferred_element_type=_f32)
                    + bout_ref[...])


_final = pl.pallas_call(
    _final_body,
    out_shape=jax.ShapeDtypeStruct((B, 1), _f32),
)


def kernel(x_mol, edge_index, graph_ids, x_adduct,
           Wg1, bg1, Wg2, bg2, Wg3, bg3,
           Wd1, bd1, Wd2, bd2, Wout, bout):
    x_pad = jnp.pad(x_mol, ((0, NP - N), (0, 0)))
    src = jnp.pad(edge_index[0], (0, E_PAD - E), constant_values=PAD_NODE)
    dst = jnp.pad(edge_index[1], (0, E_PAD - E), constant_values=PAD_NODE)
    gid = jnp.pad(graph_ids, (0, NP - N), constant_values=B)[None, :]
    zeros = jnp.zeros((NP, 128), dtype=_f32)

    dst2 = dst.reshape(E_PAD // CH, CH)
    ones2d = jnp.ones((CH, 128), dtype=_f32)
    src = src + (jnp.arange(E_PAD, dtype=jnp.int32) // CH % REP) * NP
    deg0, deg1 = _deg_kernel(dst2, ones2d, zeros)
    y1, isr = _prescale(x_pad, deg0, deg1)

    t0, t1 = _agg_kernel(src, dst, _replicate(y1), zeros)
    y2 = _layer(t0, t1, isr, Wg1, bg1[None, :])
    t0, t1 = _agg_kernel(src, dst, _replicate(y2), zeros)
    y3 = _layer(t0, t1, isr, Wg2, bg2[None, :])
    t0, t1 = _agg_kernel(src, dst, _replicate(y3), zeros)

    return _final(t0, t1, isr, Wg3, bg3[None, :], gid,
                  x_adduct.astype(_f32),
                  Wd1[:D], Wd1[D:], bd1[None, :],
                  Wd2, bd2[None, :], Wout, bout[None, :])


# speedup vs baseline: 1.7082x; 1.1620x over previous
import functools

import jax
import jax.numpy as jnp
from jax import lax
from jax.experimental import pallas as pl
from jax.experimental.pallas import tpu as pltpu
from jax.experimental.pallas import tpu_sc as plsc

N = 10000
E = 320000
D = 128
B = 64

NC = 2
NS = 16
NW = NC * NS

NP = 10112
PAD_NODE = 10000
CH = 128
NCHUNK = 80
EPW = NCHUNK * CH
E_PAD = EPW * NW
RPT = NP // NS
NB = 2
NIB = 4
C0 = 128
C1 = 2 * NCHUNK - C0
REP = 8

_mesh = plsc.VectorSubcoreMesh(core_axis_name="c", subcore_axis_name="s")

_f32 = jnp.float32


def _deg_body(dst2_hbm, ones_hbm, zeros_hbm, deg0_hbm, deg1_hbm,
              dsts, ones_rows, sem, deg_sh):
    cid = lax.axis_index("c")
    sid = lax.axis_index("s")
    wid = cid * NS + sid
    row0 = sid * RPT

    pltpu.sync_copy(ones_hbm, ones_rows)
    pltpu.sync_copy(dst2_hbm.at[pl.ds(wid * NCHUNK, NCHUNK)], dsts)
    pltpu.sync_copy(zeros_hbm.at[pl.ds(row0, RPT)],
                    deg_sh.at[pl.ds(row0, RPT)])
    plsc.subcore_barrier()

    FD = 16

    @pl.loop(0, NCHUNK // FD)
    def _grp(g):
        @pl.loop(0, FD)
        def _fire(k):
            pltpu.async_copy(ones_rows, deg_sh.at[dsts.at[g * FD + k]],
                             sem, add=True)

        @pl.loop(0, FD)
        def _drain(k):
            pltpu.make_async_copy(ones_hbm, ones_rows, sem).wait()

    plsc.subcore_barrier()

    out = (deg0_hbm, deg1_hbm)
    for c in range(NC):
        @pl.when(cid == c)
        def _():
            pltpu.sync_copy(deg_sh.at[pl.ds(row0, RPT)],
                            out[c].at[pl.ds(row0, RPT)])


_deg_kernel = functools.partial(
    pl.kernel, _deg_body, mesh=_mesh,
    out_type=[jax.ShapeDtypeStruct((NP, 128), _f32)] * 2,
    scratch_types=[
        pltpu.VMEM((NCHUNK, CH), jnp.int32),
        pltpu.VMEM((CH, 128), _f32),
        pltpu.SemaphoreType.DMA,
        pltpu.VMEM_SHARED((NP, 128), _f32),
    ],
)()


def _agg_body(src_hbm, dst_hbm, y_hbm, zeros_hbm, t0_hbm, t1_hbm,
              srcv, dstv, bufs, isems, gsems, ssems, t_sh):
    cid = lax.axis_index("c")
    sid = lax.axis_index("s")
    wid = cid * NS + sid
    row0 = sid * RPT

    def wait_idx(slot):
        pltpu.make_async_copy(src_hbm.at[pl.ds(0, CH)],
                              srcv.at[slot], isems[slot]).wait()
        pltpu.make_async_copy(src_hbm.at[pl.ds(0, CH)],
                              dstv.at[slot], isems[slot]).wait()

    def wait_buf(slot, sem):
        pltpu.make_async_copy(zeros_hbm.at[pl.ds(0, CH)],
                              bufs[slot], sem).wait()

    SUB = 4
    SR = CH // SUB

    def gather_chunk(isl, r):
        for q in range(SUB):
            pltpu.async_copy(y_hbm.at[srcv.at[isl, pl.ds(q * SR, SR)]],
                             bufs[r].at[pl.ds(q * SR, SR)], gsems[r])

    init_src = (zeros_hbm, y_hbm)
    for c in range(NC):
        @pl.when(cid == c)
        def _():
            pltpu.sync_copy(init_src[c].at[pl.ds(row0, RPT)],
                            t_sh.at[pl.ds(row0, RPT)])

    plsc.subcore_barrier()

    def run_chunks(cbase, cn):
        def load_idx(slot, j):
            off = (cbase + j) * CH
            pltpu.async_copy(src_hbm.at[pl.ds(off, CH)],
                             srcv.at[slot], isems[slot])
            pltpu.async_copy(dst_hbm.at[pl.ds(off, CH)],
                             dstv.at[slot], isems[slot])

        for s in range(NIB):
            load_idx(s, s)
        for r in range(NB):
            wait_idx(r)
            gather_chunk(r, r)

        @pl.loop(0, cn // NIB)
        def _grp(g):
            for k in range(NIB):
                j = g * NIB + k
                r = k % NB
                wait_buf(r, gsems[r])
                pltpu.async_copy(bufs[r], t_sh.at[dstv.at[k]],
                                 ssems[r], add=True)
                nj = j + NB

                @pl.when(nj < cn)
                def _():
                    wait_buf(r, ssems[r])
                    wait_idx((k + NB) % NIB)
                    gather_chunk((k + NB) % NIB, r)
                    nl = j + NIB

                    @pl.when(nl < cn)
                    def _():
                        load_idx(k, nl)

        for r in range(NB):
            wait_buf(r, ssems[r])

    if C0 > 0:
        @pl.when(cid == 0)
        def _():
            run_chunks(sid * C0, C0)

    if C1 > 0:
        @pl.when(cid == 1)
        def _():
            run_chunks(NS * C0 + sid * C1, C1)

    plsc.subcore_barrier()

    out = (t0_hbm, t1_hbm)
    for c in range(NC):
        @pl.when(cid == c)
        def _():
            pltpu.sync_copy(t_sh.at[pl.ds(row0, RPT)],
                            out[c].at[pl.ds(row0, RPT)])


_agg_kernel = functools.partial(
    pl.kernel, _agg_body, mesh=_mesh,
    out_type=[jax.ShapeDtypeStruct((NP, 128), _f32)] * 2,
    scratch_types=[
        pltpu.VMEM((NIB, CH), jnp.int32),
        pltpu.VMEM((NIB, CH), jnp.int32),
        [pltpu.VMEM((CH, 128), _f32)] * NB,
        [pltpu.SemaphoreType.DMA] * NIB,
        [pltpu.SemaphoreType.DMA] * NB,
        [pltpu.SemaphoreType.DMA] * NB,
        pltpu.VMEM_SHARED((NP, 128), _f32),
    ],
)()


def _rep_body(y_ref, out_ref):
    out_ref[...] = y_ref[...]


_replicate = pl.pallas_call(
    _rep_body,
    grid=(REP,),
    in_specs=[pl.BlockSpec((NP, 128), lambda k: (0, 0))],
    out_specs=pl.BlockSpec((NP, 128), lambda k: (k, 0)),
    out_shape=jax.ShapeDtypeStruct((REP * NP, 128), _f32),
)


def _prescale_body(x_ref, d0_ref, d1_ref, y_ref, isr_ref):
    deg = d0_ref[...] + d1_ref[...] + 1.0
    rows = lax.broadcasted_iota(jnp.int32, (NP, 128), 0)
    isr = jnp.where(rows < N, lax.rsqrt(deg), 0.0)
    isr_ref[...] = isr
    y_ref[...] = x_ref[...] * isr


_prescale = pl.pallas_call(
    _prescale_body,
    out_shape=[jax.ShapeDtypeStruct((NP, 128), _f32),
               jax.ShapeDtypeStruct((NP, 128), _f32)],
)


def _layer_body(t0_ref, t1_ref, isr_ref, w_ref, b_ref, y_ref):
    isr = isr_ref[...]
    h = (t0_ref[...] + t1_ref[...]) * isr
    x = jax.nn.relu(jnp.dot(h, w_ref[...], preferred_element_type=_f32)
                    + b_ref[...])
    y_ref[...] = x * isr


_layer = pl.pallas_call(
    _layer_body,
    out_shape=jax.ShapeDtypeStruct((NP, 128), _f32),
)


def _final_body(t0_ref, t1_ref, isr_ref, w_ref, b_ref, gid_ref, xa_ref,
                wd1a_ref, wd1b_ref, bd1_ref, wd2_ref, bd2_ref,
                wout_ref, bout_ref, out_ref):
    h = (t0_ref[...] + t1_ref[...]) * isr_ref[...]
    x3 = jax.nn.relu(jnp.dot(h, w_ref[...], preferred_element_type=_f32)
                     + b_ref[...])
    gsel = lax.broadcasted_iota(jnp.int32, (B, NP), 0)
    mask = jnp.where(gsel == gid_ref[...], 1.0, 0.0)
    r = jnp.dot(mask, x3, preferred_element_type=_f32,
                precision=lax.Precision.HIGHEST)
    h1 = jax.nn.relu(
        jnp.dot(r, wd1a_ref[...], preferred_element_type=_f32)
        + jnp.dot(xa_ref[...], wd1b_ref[...], preferred_element_type=_f32)
        + bd1_ref[...])
    h2 = jax.nn.relu(jnp.dot(h1, wd2_ref[...], preferred_element_type=_f32)
                     + bd2_ref[...])
    out_ref[...] = (jnp.dot(h2, wout_ref[...], preferred_element_type=_f32)
                    + bout_ref[...])


_final = pl.pallas_call(
    _final_body,
    out_shape=jax.ShapeDtypeStruct((B, 1), _f32),
)


def kernel(x_mol, edge_index, graph_ids, x_adduct,
           Wg1, bg1, Wg2, bg2, Wg3, bg3,
           Wd1, bd1, Wd2, bd2, Wout, bout):
    x_pad = jnp.pad(x_mol, ((0, NP - N), (0, 0)))
    src = jnp.pad(edge_index[0], (0, E_PAD - E), constant_values=PAD_NODE)
    dst = jnp.pad(edge_index[1], (0, E_PAD - E), constant_values=PAD_NODE)
    gid = jnp.pad(graph_ids, (0, NP - N), constant_values=B)[None, :]
    zeros = jnp.zeros((NP, 128), dtype=_f32)

    dst2 = dst.reshape(E_PAD // CH, CH)
    ones2d = jnp.ones((CH, 128), dtype=_f32)
    src = src + (jnp.arange(E_PAD, dtype=jnp.int32) // CH % REP) * NP
    deg0, deg1 = _deg_kernel(dst2, ones2d, zeros)
    y1, isr = _prescale(x_pad, deg0, deg1)

    t0, t1 = _agg_kernel(src, dst, _replicate(y1), zeros)
    y2 = _layer(t0, t1, isr, Wg1, bg1[None, :])
    t0, t1 = _agg_kernel(src, dst, _replicate(y2), zeros)
    y3 = _layer(t0, t1, isr, Wg2, bg2[None, :])
    t0, t1 = _agg_kernel(src, dst, _replicate(y3), zeros)

    return _final(t0, t1, isr, Wg3, bg3[None, :], gid,
                  x_adduct.astype(_f32),
                  Wd1[:D], Wd1[D:], bd1[None, :],
                  Wd2, bd2[None, :], Wout, bout[None, :])
